# baseline (speedup 1.0000x reference)
import jax
import jax.numpy as jnp
from jax import lax
from jax.experimental import pallas as pl
from jax.experimental.pallas import tpu as pltpu
from jax.experimental.pallas import tpu_sc as plsc

B = 16384
F = 26
V = 100000
NC = 2
NS = 16
NW = NC * NS
BPW = B // NW
IPW = F * BPW


def _sc_body(x_hbm, w_hbm, bias_hbm, out_hbm,
             xrm_v, spm, idx_v, vals_v, out_v, bias_v, sem, sem2):
    sid = lax.axis_index("s")
    wid = sid * NC + lax.axis_index("c")
    pltpu.sync_copy(x_hbm.at[wid], xrm_v)
    pltpu.sync_copy(bias_hbm, bias_v)

    for f in range(F):
        pltpu.async_copy(xrm_v.at[:, f], spm.at[sid, pl.ds(f * BPW, BPW)], sem2)
    pltpu.make_async_copy(w_hbm.at[pl.ds(0, IPW)], idx_v, sem2).wait()
    pltpu.sync_copy(spm.at[sid], idx_v)

    def add_off(r, carry):
        off = (r // (BPW // 16)) * V
        sl = pl.ds(r * 16, 16)
        idx_v[sl] = idx_v[sl] + off
        return carry
    lax.fori_loop(0, IPW // 16, add_off, 0)

    pltpu.async_copy(w_hbm.at[idx_v], vals_v, sem).wait()

    bias_vec = bias_v[...]
    for c in range(BPW // 16):
        acc = bias_vec
        for f in range(F):
            acc = acc + vals_v[pl.ds(f * BPW + c * 16, 16)]
        out_v[pl.ds(c * 16, 16)] = acc

    pltpu.sync_copy(out_v, out_hbm.at[wid])


def kernel(x_sparse, W, bias):
    x3 = x_sparse.astype(jnp.int32).reshape(NW, BPW, F)
    wflat = W.reshape(-1)
    bias16 = jnp.broadcast_to(bias.astype(jnp.float32), (16,))
    mesh = plsc.VectorSubcoreMesh(core_axis_name="c", subcore_axis_name="s")
    out = pl.kernel(
        _sc_body,
        out_type=jax.ShapeDtypeStruct((NW, BPW), jnp.float32),
        mesh=mesh,
        scratch_types=[
            pltpu.VMEM((BPW, F), jnp.int32),
            pltpu.VMEM_SHARED((NS, IPW), jnp.int32),
            pltpu.VMEM((IPW,), jnp.int32),
            pltpu.VMEM((IPW,), jnp.float32),
            pltpu.VMEM((BPW,), jnp.float32),
            pltpu.VMEM((16,), jnp.float32),
            pltpu.SemaphoreType.DMA,
            pltpu.SemaphoreType.DMA,
        ],
    )(x3, wflat, bias16)
    return out.reshape(B, 1)


# speedup vs baseline: 1.9250x; 1.9250x over previous
import jax
import jax.numpy as jnp
from jax import lax
from jax.experimental import pallas as pl
from jax.experimental.pallas import tpu as pltpu
from jax.experimental.pallas import tpu_sc as plsc

B = 16384
F = 26
V = 100000
NC = 2
NS = 16
NW = NC * NS
BPW = B // NW
IPW = F * BPW


def _sc_body(x_hbm, w_hbm, bias_hbm, out_hbm, idx_v, vals_v, out_v, bias_v, sem):
    wid = lax.axis_index("s") * NC + lax.axis_index("c")
    pltpu.sync_copy(x_hbm.at[wid], idx_v)
    pltpu.sync_copy(bias_hbm, bias_v)

    pltpu.async_copy(w_hbm.at[idx_v], vals_v, sem).wait()

    bias_vec = bias_v[...]
    for c in range(BPW // 16):
        acc = bias_vec
        for f in range(F):
            acc = acc + vals_v[pl.ds(f * BPW + c * 16, 16)]
        out_v[pl.ds(c * 16, 16)] = acc

    pltpu.sync_copy(out_v, out_hbm.at[wid])


def kernel(x_sparse, W, bias):
    xf = x_sparse.astype(jnp.int32) + jnp.arange(F, dtype=jnp.int32) * V
    x2 = xf.T.reshape(F, NW, BPW).transpose(1, 0, 2).reshape(NW, IPW)
    wflat = W.reshape(-1)
    bias16 = jnp.broadcast_to(bias.astype(jnp.float32), (16,))
    mesh = plsc.VectorSubcoreMesh(core_axis_name="c", subcore_axis_name="s")
    out = pl.kernel(
        _sc_body,
        out_type=jax.ShapeDtypeStruct((NW, BPW), jnp.float32),
        mesh=mesh,
        scratch_types=[
            pltpu.VMEM((IPW,), jnp.int32),
            pltpu.VMEM((IPW,), jnp.float32),
            pltpu.VMEM((BPW,), jnp.float32),
            pltpu.VMEM((16,), jnp.float32),
            pltpu.SemaphoreType.DMA,
        ],
    )(x2, wflat, bias16)
    return out.reshape(B, 1)
